# baseline (speedup 1.0000x reference)
import functools

import jax
import jax.numpy as jnp
from jax import lax
from jax.experimental import pallas as pl
from jax.experimental.pallas import tpu as pltpu
from jax.experimental.pallas import tpu_sc as plsc

B, X, Y, Z, C = 2, 96, 96, 96, 2
ZP = 128
N = X * Y * Z
TOT = B * N
NC, NS = 2, 16
PPW = X // NS
_SX, _SY = Y * Z, Z

NPEN = 12
BL = NPEN * Z
GRPS = BL // 16
NCH = BL // 128

VROWS = B * X * Y * C
DROWS = B * X * 3 * Y
OROWS = B * X * Y


def _sc_call(body, interpret=False):
    return pl.kernel(
        body,
        out_type=(
            jax.ShapeDtypeStruct((OROWS * 2 * ZP,), jnp.float32),
            jax.ShapeDtypeStruct((TOT, 8), jnp.float32),
        ),
        mesh=plsc.VectorSubcoreMesh(core_axis_name="c", subcore_axis_name="s",
                                    num_cores=NC, num_subcores=NS),
        compiler_params=pltpu.CompilerParams(needs_layout_passes=False,
                                             use_tc_tiling_on_sc=False),
        scratch_types=[
            pltpu.VMEM((98, ZP), jnp.float32),
            pltpu.VMEM((48 * Y, 8), jnp.float32),
            pltpu.VMEM((2, 3, NPEN, ZP), jnp.float32),
            pltpu.VMEM((2, 2, NCH, 128), jnp.int32),
            pltpu.VMEM((2, 8, BL), jnp.float32),
            pltpu.VMEM((2, 2, NCH, 128, 8), jnp.float32),
            pltpu.VMEM((NPEN * 2 * ZP,), jnp.float32),
            pltpu.SemaphoreType.DMA,
            pltpu.SemaphoreType.DMA,
        ],
        interpret=interpret,
    )


def _resample_body(volp, defp, out_hbm, table_hbm, stag_v, tbuf_v, coords_v,
                   idx_v, w_v, rows_v, out_v, sem, sem2):
    core = lax.axis_index("c")
    sub = lax.axis_index("s")
    batch_row0 = core * N
    lanes = lax.iota(jnp.int32, 16)

    a_row = ((lanes % 8) // 4) * 2 + (lanes % 2)
    a_col = lanes // 8 + (lanes % 4) // 2
    t_row = lanes // 8
    t_col = lanes % 8

    def build_plane(xi, _):
        x = sub * PPW + xi
        vrow0 = (core * X + x) * (Y * C)
        for yb, (y0, ny) in enumerate(((0, 48), (48, 47))):
            pltpu.sync_copy(volp.at[pl.ds(vrow0 + 2 * y0, 2 * (ny + 1)), :],
                            stag_v.at[pl.ds(0, 2 * (ny + 1)), :])

            def build_pencil(yl, _):
                rowc = a_row + 2 * yl

                trowc = t_row + yl * Z

                def build_grp(g, c2):
                    vals = plsc.load_gather(stag_v, [rowc, a_col + 2 * g])
                    plsc.store_scatter(tbuf_v, [trowc + 2 * g, t_col], vals)
                    return c2

                lax.fori_loop(0, Z // 2, build_grp, 0)
                return _

            lax.fori_loop(0, ny, build_pencil, 0)
            t0 = batch_row0 + x * _SX + y0 * _SY
            pltpu.sync_copy(tbuf_v.at[pl.ds(0, ny * Z), :],
                            table_hbm.at[pl.ds(t0, ny * Z), :])
        return _

    lax.fori_loop(0, PPW, build_plane, 0)
    plsc.subcore_barrier()

    zero = jnp.zeros((16,), jnp.int32)
    NBLK = PPW * (Y // NPEN)

    def blk_xy(t):
        x = sub * PPW + t // (Y // NPEN)
        y0 = (t % (Y // NPEN)) * NPEN
        return x, y0

    def stage_coords(t, pb):
        x, y0 = blk_xy(t)
        drow0 = (core * X + x) * (3 * Y) + y0
        for comp in range(3):
            pltpu.sync_copy(defp.at[pl.ds(drow0 + comp * Y, NPEN), :],
                            coords_v.at[pb, comp])

    def compute_idx(pb):
        def grp_index(g, c2):
            yl = g // (Z // 16)
            zoff = (g % (Z // 16)) * 16
            xs = coords_v[pb, 0, yl, pl.ds(zoff, 16)]
            ys = coords_v[pb, 1, yl, pl.ds(zoff, 16)]
            zs = coords_v[pb, 2, yl, pl.ds(zoff, 16)]
            ix = jnp.minimum(xs.astype(jnp.int32), X - 2)
            iy = jnp.minimum(ys.astype(jnp.int32), Y - 2)
            iz = jnp.minimum(zs.astype(jnp.int32), Z - 2)
            fx = xs - ix.astype(jnp.float32)
            fy = ys - iy.astype(jnp.float32)
            fz = zs - iz.astype(jnp.float32)
            gx = 1.0 - fx
            gy = 1.0 - fy
            gz = 1.0 - fz
            v = batch_row0 + ix * _SX + iy * _SY + iz
            jj = g // 8
            pos = (g % 8) * 16
            idx_v[pb, 0, jj, pl.ds(pos, 16)] = v
            idx_v[pb, 1, jj, pl.ds(pos, 16)] = v + _SX
            wyz = (gy * gz, gy * fz, fy * gz, fy * fz)
            b16 = g * 16
            for dy in (0, 1):
                for dz in (0, 1):
                    w_v[pb, dy * 2 + dz, pl.ds(b16, 16)] = gx * wyz[dy * 2 + dz]
                    w_v[pb, 4 + dy * 2 + dz, pl.ds(b16, 16)] = fx * wyz[dy * 2 + dz]
            return c2

        lax.fori_loop(0, GRPS, grp_index, 0)

    def fire(pb, s):
        for dx in range(2):
            for jj in range(NCH):
                pltpu.async_copy(table_hbm.at[idx_v.at[pb, dx, jj]],
                                 rows_v.at[pb, dx, jj], s)

    def drain(pb, s):
        for dx in range(2):
            for jj in range(NCH):
                pltpu.make_async_copy(table_hbm.at[idx_v.at[pb, dx, jj]],
                                      rows_v.at[pb, dx, jj], s).wait()

    def extract(t, pb):
        def grp_acc(g, c2):
            yl = g // (Z // 16)
            zoff = (g % (Z // 16)) * 16
            jj = g // 8
            posv = (g % 8) * 16 + lanes
            b16 = g * 16
            acc0 = jnp.zeros((16,), jnp.float32)
            acc1 = jnp.zeros((16,), jnp.float32)
            for dx in range(2):
                rv = rows_v.at[pb, dx, jj]
                for dy in range(2):
                    for dz in range(2):
                        w = w_v[pb, dx * 4 + dy * 2 + dz, pl.ds(b16, 16)]
                        f = dy * 4 + dz * 2
                        v0 = plsc.load_gather(rv, [posv, zero + f])
                        v1 = plsc.load_gather(rv, [posv, zero + f + 1])
                        acc0 = acc0 + w * v0
                        acc1 = acc1 + w * v1
            out_v[pl.ds(yl * (2 * ZP) + zoff, 16)] = acc0
            out_v[pl.ds(yl * (2 * ZP) + ZP + zoff, 16)] = acc1
            return c2

        lax.fori_loop(0, GRPS, grp_acc, 0)
        x, y0 = blk_xy(t)
        o0 = ((core * X + x) * Y + y0) * (2 * ZP)
        pltpu.sync_copy(out_v, out_hbm.at[pl.ds(o0, NPEN * 2 * ZP)])

    stage_coords(0, 0)
    compute_idx(0)
    fire(0, sem)

    def pair(tt, _):
        t0 = tt * 2
        stage_coords(t0 + 1, 1)
        compute_idx(1)
        fire(1, sem2)
        drain(0, sem)
        extract(t0, 0)

        @pl.when(t0 + 2 < NBLK)
        def _prep():
            stage_coords(t0 + 2, 0)
            compute_idx(0)
            fire(0, sem)

        drain(1, sem2)
        extract(t0 + 1, 1)
        return _

    lax.fori_loop(0, NBLK // 2, pair, 0)


_resample_sc = _sc_call(_resample_body)


def kernel(inputs, deformation):
    volp = jnp.pad(
        jnp.transpose(inputs, (0, 1, 2, 4, 3)).reshape(VROWS, Z),
        ((0, 0), (0, ZP - Z)))
    defp = jnp.pad(
        jnp.transpose(deformation, (0, 1, 4, 2, 3)).reshape(DROWS, Z),
        ((0, 0), (0, ZP - Z)))
    out_flat, _ = _resample_sc(volp, defp)
    out = out_flat.reshape(B, X, Y, C, ZP)[..., :Z]
    return jnp.transpose(out, (0, 1, 2, 4, 3))


# speedup vs baseline: 1.0017x; 1.0017x over previous
import functools

import jax
import jax.numpy as jnp
from jax import lax
from jax.experimental import pallas as pl
from jax.experimental.pallas import tpu as pltpu
from jax.experimental.pallas import tpu_sc as plsc

B, X, Y, Z, C = 2, 96, 96, 96, 2
ZP = 128
N = X * Y * Z
TOT = B * N
NC, NS = 2, 16
PPW = X // NS
_SX, _SY = Y * Z, Z

NPEN = 12
BL = NPEN * Z
GRPS = BL // 16
NCH = BL // 128

VROWS = B * X * Y * C
DROWS = B * X * 3 * Y
OROWS = B * X * Y


def _sc_call(body, interpret=False):
    return pl.kernel(
        body,
        out_type=(
            jax.ShapeDtypeStruct((OROWS * 2 * ZP,), jnp.float32),
            jax.ShapeDtypeStruct((TOT, 8), jnp.float32),
        ),
        mesh=plsc.VectorSubcoreMesh(core_axis_name="c", subcore_axis_name="s",
                                    num_cores=NC, num_subcores=NS),
        compiler_params=pltpu.CompilerParams(needs_layout_passes=False,
                                             use_tc_tiling_on_sc=False),
        scratch_types=[
            pltpu.VMEM((98, ZP), jnp.float32),
            pltpu.VMEM((48 * Y, 8), jnp.float32),
            pltpu.VMEM((2, 3, NPEN, ZP), jnp.float32),
            pltpu.VMEM((2, 2, BL), jnp.int32),
            pltpu.VMEM((2, 8, BL), jnp.float32),
            pltpu.VMEM((2, 2, BL, 8), jnp.float32),
            pltpu.VMEM((NPEN * 2 * ZP,), jnp.float32),
            pltpu.SemaphoreType.DMA,
            pltpu.SemaphoreType.DMA,
        ],
        interpret=interpret,
    )


def _resample_body(volp, defp, out_hbm, table_hbm, stag_v, tbuf_v, coords_v,
                   idx_v, w_v, rows_v, out_v, sem, sem2):
    core = lax.axis_index("c")
    sub = lax.axis_index("s")
    batch_row0 = core * N
    lanes = lax.iota(jnp.int32, 16)

    a_row = ((lanes % 8) // 4) * 2 + (lanes % 2)
    a_col = lanes // 8 + (lanes % 4) // 2
    t_row = lanes // 8
    t_col = lanes % 8

    def build_plane(xi, _):
        x = sub * PPW + xi
        vrow0 = (core * X + x) * (Y * C)
        for yb, (y0, ny) in enumerate(((0, 48), (48, 47))):
            pltpu.sync_copy(volp.at[pl.ds(vrow0 + 2 * y0, 2 * (ny + 1)), :],
                            stag_v.at[pl.ds(0, 2 * (ny + 1)), :])

            def build_pencil(yl, _):
                rowc = a_row + 2 * yl

                trowc = t_row + yl * Z

                def build_grp(g, c2):
                    vals = plsc.load_gather(stag_v, [rowc, a_col + 2 * g])
                    plsc.store_scatter(tbuf_v, [trowc + 2 * g, t_col], vals)
                    return c2

                lax.fori_loop(0, Z // 2, build_grp, 0)
                return _

            lax.fori_loop(0, ny, build_pencil, 0)
            t0 = batch_row0 + x * _SX + y0 * _SY
            pltpu.sync_copy(tbuf_v.at[pl.ds(0, ny * Z), :],
                            table_hbm.at[pl.ds(t0, ny * Z), :])
        return _

    lax.fori_loop(0, PPW, build_plane, 0)
    plsc.subcore_barrier()

    zero = jnp.zeros((16,), jnp.int32)
    NBLK = PPW * (Y // NPEN)

    def blk_xy(t):
        x = sub * PPW + t // (Y // NPEN)
        y0 = (t % (Y // NPEN)) * NPEN
        return x, y0

    def stage_coords(t, pb):
        x, y0 = blk_xy(t)
        drow0 = (core * X + x) * (3 * Y) + y0
        for comp in range(3):
            pltpu.sync_copy(defp.at[pl.ds(drow0 + comp * Y, NPEN), :],
                            coords_v.at[pb, comp])

    def compute_idx(pb):
        def grp_index(g, c2):
            yl = g // (Z // 16)
            zoff = (g % (Z // 16)) * 16
            xs = coords_v[pb, 0, yl, pl.ds(zoff, 16)]
            ys = coords_v[pb, 1, yl, pl.ds(zoff, 16)]
            zs = coords_v[pb, 2, yl, pl.ds(zoff, 16)]
            ix = jnp.minimum(xs.astype(jnp.int32), X - 2)
            iy = jnp.minimum(ys.astype(jnp.int32), Y - 2)
            iz = jnp.minimum(zs.astype(jnp.int32), Z - 2)
            fx = xs - ix.astype(jnp.float32)
            fy = ys - iy.astype(jnp.float32)
            fz = zs - iz.astype(jnp.float32)
            gx = 1.0 - fx
            gy = 1.0 - fy
            gz = 1.0 - fz
            v = batch_row0 + ix * _SX + iy * _SY + iz
            b16 = g * 16
            idx_v[pb, 0, pl.ds(b16, 16)] = v
            idx_v[pb, 1, pl.ds(b16, 16)] = v + _SX
            wyz = (gy * gz, gy * fz, fy * gz, fy * fz)
            for dy in (0, 1):
                for dz in (0, 1):
                    w_v[pb, dy * 2 + dz, pl.ds(b16, 16)] = gx * wyz[dy * 2 + dz]
                    w_v[pb, 4 + dy * 2 + dz, pl.ds(b16, 16)] = fx * wyz[dy * 2 + dz]
            return c2

        lax.fori_loop(0, GRPS, grp_index, 0)

    def fire(pb, s):
        for dx in range(2):
            pltpu.async_copy(table_hbm.at[idx_v.at[pb, dx]],
                             rows_v.at[pb, dx], s)

    def drain(pb, s):
        for dx in range(2):
            pltpu.make_async_copy(table_hbm.at[idx_v.at[pb, dx]],
                                  rows_v.at[pb, dx], s).wait()

    def extract(t, pb):
        def grp_acc(g, c2):
            yl = g // (Z // 16)
            zoff = (g % (Z // 16)) * 16
            b16 = g * 16
            posv = b16 + lanes
            acc0 = jnp.zeros((16,), jnp.float32)
            acc1 = jnp.zeros((16,), jnp.float32)
            for dx in range(2):
                rv = rows_v.at[pb, dx]
                for dy in range(2):
                    for dz in range(2):
                        w = w_v[pb, dx * 4 + dy * 2 + dz, pl.ds(b16, 16)]
                        f = dy * 4 + dz * 2
                        v0 = plsc.load_gather(rv, [posv, zero + f])
                        v1 = plsc.load_gather(rv, [posv, zero + f + 1])
                        acc0 = acc0 + w * v0
                        acc1 = acc1 + w * v1
            out_v[pl.ds(yl * (2 * ZP) + zoff, 16)] = acc0
            out_v[pl.ds(yl * (2 * ZP) + ZP + zoff, 16)] = acc1
            return c2

        lax.fori_loop(0, GRPS, grp_acc, 0)
        x, y0 = blk_xy(t)
        o0 = ((core * X + x) * Y + y0) * (2 * ZP)
        pltpu.sync_copy(out_v, out_hbm.at[pl.ds(o0, NPEN * 2 * ZP)])

    stage_coords(0, 0)
    compute_idx(0)
    fire(0, sem)

    def pair(tt, _):
        t0 = tt * 2
        stage_coords(t0 + 1, 1)
        compute_idx(1)
        fire(1, sem2)
        drain(0, sem)
        extract(t0, 0)

        @pl.when(t0 + 2 < NBLK)
        def _prep():
            stage_coords(t0 + 2, 0)
            compute_idx(0)
            fire(0, sem)

        drain(1, sem2)
        extract(t0 + 1, 1)
        return _

    lax.fori_loop(0, NBLK // 2, pair, 0)


_resample_sc = _sc_call(_resample_body)


def kernel(inputs, deformation):
    volp = jnp.pad(
        jnp.transpose(inputs, (0, 1, 2, 4, 3)).reshape(VROWS, Z),
        ((0, 0), (0, ZP - Z)))
    defp = jnp.pad(
        jnp.transpose(deformation, (0, 1, 4, 2, 3)).reshape(DROWS, Z),
        ((0, 0), (0, ZP - Z)))
    out_flat, _ = _resample_sc(volp, defp)
    out = out_flat.reshape(B, X, Y, C, ZP)[..., :Z]
    return jnp.transpose(out, (0, 1, 2, 4, 3))


# speedup vs baseline: 1.0383x; 1.0366x over previous
import functools

import jax
import jax.numpy as jnp
from jax import lax
from jax.experimental import pallas as pl
from jax.experimental.pallas import tpu as pltpu
from jax.experimental.pallas import tpu_sc as plsc

B, X, Y, Z, C = 2, 96, 96, 96, 2
ZP = 128
N = X * Y * Z
TOT = B * N
NC, NS = 2, 16
PPW = X // NS
_SX, _SY = Y * Z, Z

NPEN = 12
BL = NPEN * Z
GRPS = BL // 16
NCH = BL // 128

VROWS = B * X * Y * C
DROWS = B * X * 3 * Y
OROWS = B * X * Y


def _sc_call(body, interpret=False):
    return pl.kernel(
        body,
        out_type=(
            jax.ShapeDtypeStruct((OROWS * 2 * ZP,), jnp.float32),
            jax.ShapeDtypeStruct((TOT, 8), jnp.float32),
        ),
        mesh=plsc.VectorSubcoreMesh(core_axis_name="c", subcore_axis_name="s",
                                    num_cores=NC, num_subcores=NS),
        compiler_params=pltpu.CompilerParams(needs_layout_passes=False,
                                             use_tc_tiling_on_sc=False),
        scratch_types=[
            pltpu.VMEM((98, ZP), jnp.float32),
            pltpu.VMEM((48 * Y, 8), jnp.float32),
            pltpu.VMEM((2, 3, NPEN, ZP), jnp.float32),
            pltpu.VMEM((2, 2, BL), jnp.int32),
            pltpu.VMEM((2, 8, BL), jnp.float32),
            pltpu.VMEM((2, 2, BL, 8), jnp.float32),
            pltpu.VMEM((NPEN * 2 * ZP,), jnp.float32),
            pltpu.SemaphoreType.DMA,
            pltpu.SemaphoreType.DMA,
        ],
        interpret=interpret,
    )


def _resample_body(volp, defp, out_hbm, table_hbm, stag_v, tbuf_v, coords_v,
                   idx_v, w_v, rows_v, out_v, sem, sem2):
    core = lax.axis_index("c")
    sub = lax.axis_index("s")
    batch_row0 = core * N
    lanes = lax.iota(jnp.int32, 16)

    a_row = ((lanes % 8) // 4) * 2 + (lanes % 2)
    a_col = lanes // 8 + (lanes % 4) // 2
    t_row = lanes // 8
    t_col = lanes % 8

    def build_plane(xi, _):
        x = sub * PPW + xi
        vrow0 = (core * X + x) * (Y * C)
        for yb, (y0, ny) in enumerate(((0, 48), (48, 47))):
            pltpu.sync_copy(volp.at[pl.ds(vrow0 + 2 * y0, 2 * (ny + 1)), :],
                            stag_v.at[pl.ds(0, 2 * (ny + 1)), :])

            def build_pencil(yl, _):
                rowc = a_row + 2 * yl

                trowc = t_row + yl * Z

                def build_grp(i, c2):
                    for u in range(4):
                        g = i * 4 + u
                        vals = plsc.load_gather(stag_v, [rowc, a_col + 2 * g])
                        plsc.store_scatter(tbuf_v, [trowc + 2 * g, t_col], vals)
                    return c2

                lax.fori_loop(0, Z // 8, build_grp, 0)
                return _

            lax.fori_loop(0, ny, build_pencil, 0)
            t0 = batch_row0 + x * _SX + y0 * _SY
            pltpu.sync_copy(tbuf_v.at[pl.ds(0, ny * Z), :],
                            table_hbm.at[pl.ds(t0, ny * Z), :])
        return _

    lax.fori_loop(0, PPW, build_plane, 0)
    plsc.subcore_barrier()

    zero = jnp.zeros((16,), jnp.int32)
    NBLK = PPW * (Y // NPEN)

    def blk_xy(t):
        x = sub * PPW + t // (Y // NPEN)
        y0 = (t % (Y // NPEN)) * NPEN
        return x, y0

    def stage_coords(t, pb):
        x, y0 = blk_xy(t)
        drow0 = (core * X + x) * (3 * Y) + y0
        for comp in range(3):
            pltpu.sync_copy(defp.at[pl.ds(drow0 + comp * Y, NPEN), :],
                            coords_v.at[pb, comp])

    def compute_idx(pb):
        def grp_index(g, c2):
            yl = g // (Z // 16)
            zoff = (g % (Z // 16)) * 16
            xs = coords_v[pb, 0, yl, pl.ds(zoff, 16)]
            ys = coords_v[pb, 1, yl, pl.ds(zoff, 16)]
            zs = coords_v[pb, 2, yl, pl.ds(zoff, 16)]
            ix = jnp.minimum(xs.astype(jnp.int32), X - 2)
            iy = jnp.minimum(ys.astype(jnp.int32), Y - 2)
            iz = jnp.minimum(zs.astype(jnp.int32), Z - 2)
            fx = xs - ix.astype(jnp.float32)
            fy = ys - iy.astype(jnp.float32)
            fz = zs - iz.astype(jnp.float32)
            gx = 1.0 - fx
            gy = 1.0 - fy
            gz = 1.0 - fz
            v = batch_row0 + ix * _SX + iy * _SY + iz
            b16 = g * 16
            idx_v[pb, 0, pl.ds(b16, 16)] = v
            idx_v[pb, 1, pl.ds(b16, 16)] = v + _SX
            wyz = (gy * gz, gy * fz, fy * gz, fy * fz)
            for dy in (0, 1):
                for dz in (0, 1):
                    w_v[pb, dy * 2 + dz, pl.ds(b16, 16)] = gx * wyz[dy * 2 + dz]
                    w_v[pb, 4 + dy * 2 + dz, pl.ds(b16, 16)] = fx * wyz[dy * 2 + dz]
            return c2

        def grp_index2(i, c2):
            grp_index(i * 2, c2)
            grp_index(i * 2 + 1, c2)
            return c2

        lax.fori_loop(0, GRPS // 2, grp_index2, 0)

    def fire(pb, s):
        for dx in range(2):
            pltpu.async_copy(table_hbm.at[idx_v.at[pb, dx]],
                             rows_v.at[pb, dx], s)

    def drain(pb, s):
        for dx in range(2):
            pltpu.make_async_copy(table_hbm.at[idx_v.at[pb, dx]],
                                  rows_v.at[pb, dx], s).wait()

    def extract(t, pb):
        def grp_acc(g, c2):
            yl = g // (Z // 16)
            zoff = (g % (Z // 16)) * 16
            b16 = g * 16
            posv = b16 + lanes
            acc0 = jnp.zeros((16,), jnp.float32)
            acc1 = jnp.zeros((16,), jnp.float32)
            for dx in range(2):
                rv = rows_v.at[pb, dx]
                for dy in range(2):
                    for dz in range(2):
                        w = w_v[pb, dx * 4 + dy * 2 + dz, pl.ds(b16, 16)]
                        f = dy * 4 + dz * 2
                        v0 = plsc.load_gather(rv, [posv, zero + f])
                        v1 = plsc.load_gather(rv, [posv, zero + f + 1])
                        acc0 = acc0 + w * v0
                        acc1 = acc1 + w * v1
            out_v[pl.ds(yl * (2 * ZP) + zoff, 16)] = acc0
            out_v[pl.ds(yl * (2 * ZP) + ZP + zoff, 16)] = acc1
            return c2

        def grp_acc2(i, c2):
            grp_acc(i * 2, c2)
            grp_acc(i * 2 + 1, c2)
            return c2

        lax.fori_loop(0, GRPS // 2, grp_acc2, 0)
        x, y0 = blk_xy(t)
        o0 = ((core * X + x) * Y + y0) * (2 * ZP)
        pltpu.sync_copy(out_v, out_hbm.at[pl.ds(o0, NPEN * 2 * ZP)])

    stage_coords(0, 0)
    compute_idx(0)
    fire(0, sem)

    def pair(tt, _):
        t0 = tt * 2
        stage_coords(t0 + 1, 1)
        compute_idx(1)
        fire(1, sem2)
        drain(0, sem)
        extract(t0, 0)

        @pl.when(t0 + 2 < NBLK)
        def _prep():
            stage_coords(t0 + 2, 0)
            compute_idx(0)
            fire(0, sem)

        drain(1, sem2)
        extract(t0 + 1, 1)
        return _

    lax.fori_loop(0, NBLK // 2, pair, 0)


_resample_sc = _sc_call(_resample_body)


def kernel(inputs, deformation):
    volp = jnp.pad(
        jnp.transpose(inputs, (0, 1, 2, 4, 3)).reshape(VROWS, Z),
        ((0, 0), (0, ZP - Z)))
    defp = jnp.pad(
        jnp.transpose(deformation, (0, 1, 4, 2, 3)).reshape(DROWS, Z),
        ((0, 0), (0, ZP - Z)))
    out_flat, _ = _resample_sc(volp, defp)
    out = out_flat.reshape(B, X, Y, C, ZP)[..., :Z]
    return jnp.transpose(out, (0, 1, 2, 4, 3))


# speedup vs baseline: 1.0422x; 1.0037x over previous
import functools

import jax
import jax.numpy as jnp
from jax import lax
from jax.experimental import pallas as pl
from jax.experimental.pallas import tpu as pltpu
from jax.experimental.pallas import tpu_sc as plsc

B, X, Y, Z, C = 2, 96, 96, 96, 2
ZP = 128
N = X * Y * Z
TOT = B * N
NC, NS = 2, 16
PPW = X // NS
_SX, _SY = Y * Z, Z

NPEN = 12
BL = NPEN * Z
GRPS = BL // 16
NCH = BL // 128

VROWS = B * X * Y * C
DROWS = B * X * 3 * Y
OROWS = B * X * Y


def _sc_call(body, interpret=False):
    return pl.kernel(
        body,
        out_type=(
            jax.ShapeDtypeStruct((OROWS * 2 * ZP,), jnp.float32),
            jax.ShapeDtypeStruct((TOT, 8), jnp.float32),
        ),
        mesh=plsc.VectorSubcoreMesh(core_axis_name="c", subcore_axis_name="s",
                                    num_cores=NC, num_subcores=NS),
        compiler_params=pltpu.CompilerParams(needs_layout_passes=False,
                                             use_tc_tiling_on_sc=False),
        scratch_types=[
            pltpu.VMEM((50, ZP), jnp.float32),
            pltpu.VMEM((2, 24 * Z, 8), jnp.float32),
            pltpu.VMEM((2, 3, NPEN, ZP), jnp.float32),
            pltpu.VMEM((2, 2, BL), jnp.int32),
            pltpu.VMEM((2, 8, BL), jnp.float32),
            pltpu.VMEM((2, 2, BL, 8), jnp.float32),
            pltpu.VMEM((NPEN * 2 * ZP,), jnp.float32),
            pltpu.SemaphoreType.DMA,
            pltpu.SemaphoreType.DMA,
            pltpu.SemaphoreType.DMA,
            pltpu.SemaphoreType.DMA,
        ],
        interpret=interpret,
    )


def _resample_body(volp, defp, out_hbm, table_hbm, stag_v, tbuf_v, coords_v,
                   idx_v, w_v, rows_v, out_v, sem, sem2, semta, semtb):
    core = lax.axis_index("c")
    sub = lax.axis_index("s")
    batch_row0 = core * N
    lanes = lax.iota(jnp.int32, 16)

    a_row = ((lanes % 8) // 4) * 2 + (lanes % 2)
    a_col = lanes // 8 + (lanes % 4) // 2
    t_row = lanes // 8
    t_col = lanes % 8

    def build_plane(xi, _):
        x = sub * PPW + xi
        vrow0 = (core * X + x) * (Y * C)
        for k in range(4):
            y0 = 24 * k
            tb = k % 2
            semt = (semta, semtb)[tb]
            nstage = 50 if k < 3 else 48
            pltpu.sync_copy(volp.at[pl.ds(vrow0 + 48 * k, nstage), :],
                            stag_v.at[pl.ds(0, nstage), :])

            def _wait_tb():
                pltpu.make_async_copy(
                    tbuf_v.at[tb], table_hbm.at[pl.ds(0, 24 * Z), :],
                    semt).wait()

            if k < 2:
                @pl.when(xi > 0)
                def _d():
                    _wait_tb()
            else:
                _wait_tb()

            def build_pencil(yl, _):
                rowc = a_row + 2 * yl
                trowc = t_row + yl * Z

                def build_grp(i, c2):
                    for u in range(4):
                        g = i * 4 + u
                        vals = plsc.load_gather(stag_v, [rowc, a_col + 2 * g])
                        plsc.store_scatter(tbuf_v.at[tb],
                                           [trowc + 2 * g, t_col], vals)
                    return c2

                lax.fori_loop(0, Z // 8, build_grp, 0)
                return _

            lax.fori_loop(0, 24, build_pencil, 0)
            t0 = batch_row0 + x * _SX + y0 * _SY
            pltpu.async_copy(tbuf_v.at[tb],
                             table_hbm.at[pl.ds(t0, 24 * Z), :], semt)
        return _

    lax.fori_loop(0, PPW, build_plane, 0)
    for tb in range(2):
        pltpu.make_async_copy(tbuf_v.at[tb],
                              table_hbm.at[pl.ds(0, 24 * Z), :],
                              (semta, semtb)[tb]).wait()
    plsc.subcore_barrier()

    zero = jnp.zeros((16,), jnp.int32)
    NBLK = PPW * (Y // NPEN)

    def blk_xy(t):
        x = sub * PPW + t // (Y // NPEN)
        y0 = (t % (Y // NPEN)) * NPEN
        return x, y0

    def stage_coords(t, pb):
        x, y0 = blk_xy(t)
        drow0 = (core * X + x) * (3 * Y) + y0
        for comp in range(3):
            pltpu.sync_copy(defp.at[pl.ds(drow0 + comp * Y, NPEN), :],
                            coords_v.at[pb, comp])

    def compute_idx(pb):
        def grp_index(g, c2):
            yl = g // (Z // 16)
            zoff = (g % (Z // 16)) * 16
            xs = coords_v[pb, 0, yl, pl.ds(zoff, 16)]
            ys = coords_v[pb, 1, yl, pl.ds(zoff, 16)]
            zs = coords_v[pb, 2, yl, pl.ds(zoff, 16)]
            ix = jnp.minimum(xs.astype(jnp.int32), X - 2)
            iy = jnp.minimum(ys.astype(jnp.int32), Y - 2)
            iz = jnp.minimum(zs.astype(jnp.int32), Z - 2)
            fx = xs - ix.astype(jnp.float32)
            fy = ys - iy.astype(jnp.float32)
            fz = zs - iz.astype(jnp.float32)
            gx = 1.0 - fx
            gy = 1.0 - fy
            gz = 1.0 - fz
            v = batch_row0 + ix * _SX + iy * _SY + iz
            b16 = g * 16
            idx_v[pb, 0, pl.ds(b16, 16)] = v
            idx_v[pb, 1, pl.ds(b16, 16)] = v + _SX
            wyz = (gy * gz, gy * fz, fy * gz, fy * fz)
            for dy in (0, 1):
                for dz in (0, 1):
                    w_v[pb, dy * 2 + dz, pl.ds(b16, 16)] = gx * wyz[dy * 2 + dz]
                    w_v[pb, 4 + dy * 2 + dz, pl.ds(b16, 16)] = fx * wyz[dy * 2 + dz]
            return c2

        def grp_index2(i, c2):
            grp_index(i * 2, c2)
            grp_index(i * 2 + 1, c2)
            return c2

        lax.fori_loop(0, GRPS // 2, grp_index2, 0)

    def fire(pb, s):
        for dx in range(2):
            pltpu.async_copy(table_hbm.at[idx_v.at[pb, dx]],
                             rows_v.at[pb, dx], s)

    def drain(pb, s):
        for dx in range(2):
            pltpu.make_async_copy(table_hbm.at[idx_v.at[pb, dx]],
                                  rows_v.at[pb, dx], s).wait()

    def extract(t, pb):
        def grp_acc(g, c2):
            yl = g // (Z // 16)
            zoff = (g % (Z // 16)) * 16
            b16 = g * 16
            posv = b16 + lanes
            acc0 = jnp.zeros((16,), jnp.float32)
            acc1 = jnp.zeros((16,), jnp.float32)
            for dx in range(2):
                rv = rows_v.at[pb, dx]
                for dy in range(2):
                    for dz in range(2):
                        w = w_v[pb, dx * 4 + dy * 2 + dz, pl.ds(b16, 16)]
                        f = dy * 4 + dz * 2
                        v0 = plsc.load_gather(rv, [posv, zero + f])
                        v1 = plsc.load_gather(rv, [posv, zero + f + 1])
                        acc0 = acc0 + w * v0
                        acc1 = acc1 + w * v1
            out_v[pl.ds(yl * (2 * ZP) + zoff, 16)] = acc0
            out_v[pl.ds(yl * (2 * ZP) + ZP + zoff, 16)] = acc1
            return c2

        def grp_acc2(i, c2):
            grp_acc(i * 2, c2)
            grp_acc(i * 2 + 1, c2)
            return c2

        lax.fori_loop(0, GRPS // 2, grp_acc2, 0)
        x, y0 = blk_xy(t)
        o0 = ((core * X + x) * Y + y0) * (2 * ZP)
        pltpu.sync_copy(out_v, out_hbm.at[pl.ds(o0, NPEN * 2 * ZP)])

    stage_coords(0, 0)
    compute_idx(0)
    fire(0, sem)

    def pair(tt, _):
        t0 = tt * 2
        stage_coords(t0 + 1, 1)
        compute_idx(1)
        fire(1, sem2)
        drain(0, sem)
        extract(t0, 0)

        @pl.when(t0 + 2 < NBLK)
        def _prep():
            stage_coords(t0 + 2, 0)
            compute_idx(0)
            fire(0, sem)

        drain(1, sem2)
        extract(t0 + 1, 1)
        return _

    lax.fori_loop(0, NBLK // 2, pair, 0)


_resample_sc = _sc_call(_resample_body)


def kernel(inputs, deformation):
    volp = jnp.pad(
        jnp.transpose(inputs, (0, 1, 2, 4, 3)).reshape(VROWS, Z),
        ((0, 0), (0, ZP - Z)))
    defp = jnp.pad(
        jnp.transpose(deformation, (0, 1, 4, 2, 3)).reshape(DROWS, Z),
        ((0, 0), (0, ZP - Z)))
    out_flat, _ = _resample_sc(volp, defp)
    out = out_flat.reshape(B, X, Y, C, ZP)[..., :Z]
    return jnp.transpose(out, (0, 1, 2, 4, 3))


# speedup vs baseline: 1.1765x; 1.1289x over previous
import functools

import jax
import jax.numpy as jnp
from jax import lax
from jax.experimental import pallas as pl
from jax.experimental.pallas import tpu as pltpu
from jax.experimental.pallas import tpu_sc as plsc

B, X, Y, Z, C = 2, 96, 96, 96, 2
ZP = 128
N = X * Y * Z
TOT = B * N
NC, NS = 2, 16
PPW = X // NS
_SX, _SY = Y * Z, Z

NPEN = 12
BL = NPEN * Z
GRPS = BL // 16
NCH = BL // 128

VROWS = B * X * Y * C
DROWS = B * X * 3 * Y
OROWS = B * X * Y


def _sc_call(body, interpret=False):
    return pl.kernel(
        body,
        out_type=(
            jax.ShapeDtypeStruct((OROWS * 2 * ZP,), jnp.float32),
            jax.ShapeDtypeStruct((TOT, 8), jnp.float32),
        ),
        mesh=plsc.VectorSubcoreMesh(core_axis_name="c", subcore_axis_name="s",
                                    num_cores=NC, num_subcores=NS),
        compiler_params=pltpu.CompilerParams(needs_layout_passes=False,
                                             use_tc_tiling_on_sc=False),
        scratch_types=[
            pltpu.VMEM((50, ZP), jnp.float32),
            pltpu.VMEM((2, 24 * Z, 8), jnp.float32),
            pltpu.VMEM((3, 48, ZP), jnp.float32),
            pltpu.VMEM((2, 2, BL), jnp.int32),
            pltpu.VMEM((2, 8, BL), jnp.float32),
            pltpu.VMEM((2, 2, BL, 8), jnp.float32),
            pltpu.VMEM((NPEN * 2 * ZP,), jnp.float32),
            pltpu.SemaphoreType.DMA,
            pltpu.SemaphoreType.DMA,
            pltpu.SemaphoreType.DMA,
            pltpu.SemaphoreType.DMA,
        ],
        interpret=interpret,
    )


def _resample_body(volp, defp, out_hbm, table_hbm, stag_v, tbuf_v, coords_v,
                   idx_v, w_v, rows_v, out_v, sem, sem2, semta, semtb):
    core = lax.axis_index("c")
    sub = lax.axis_index("s")
    batch_row0 = core * N
    lanes = lax.iota(jnp.int32, 16)

    a_row = ((lanes % 8) // 4) * 2 + (lanes % 2)
    a_col = lanes // 8 + (lanes % 4) // 2
    t_row = lanes // 8
    t_col = lanes % 8

    def build_plane(xi, _):
        x = sub * PPW + xi
        vrow0 = (core * X + x) * (Y * C)
        for k in range(4):
            y0 = 24 * k
            tb = k % 2
            semt = (semta, semtb)[tb]
            nstage = 50 if k < 3 else 48
            pltpu.sync_copy(volp.at[pl.ds(vrow0 + 48 * k, nstage), :],
                            stag_v.at[pl.ds(0, nstage), :])

            def _wait_tb():
                pltpu.make_async_copy(
                    tbuf_v.at[tb], table_hbm.at[pl.ds(0, 24 * Z), :],
                    semt).wait()

            if k < 2:
                @pl.when(xi > 0)
                def _d():
                    _wait_tb()
            else:
                _wait_tb()

            def build_pencil(yl, _):
                rowc = a_row + 2 * yl
                trowc = t_row + yl * Z

                def build_grp(i, c2):
                    for u in range(4):
                        g = i * 4 + u
                        vals = plsc.load_gather(stag_v, [rowc, a_col + 2 * g])
                        plsc.store_scatter(tbuf_v.at[tb],
                                           [trowc + 2 * g, t_col], vals)
                    return c2

                lax.fori_loop(0, Z // 8, build_grp, 0)
                return _

            lax.fori_loop(0, 24, build_pencil, 0)
            t0 = batch_row0 + x * _SX + y0 * _SY
            pltpu.async_copy(tbuf_v.at[tb],
                             table_hbm.at[pl.ds(t0, 24 * Z), :], semt)
        return _

    lax.fori_loop(0, PPW, build_plane, 0)
    for tb in range(2):
        pltpu.make_async_copy(tbuf_v.at[tb],
                              table_hbm.at[pl.ds(0, 24 * Z), :],
                              (semta, semtb)[tb]).wait()
    plsc.subcore_barrier()

    zero = jnp.zeros((16,), jnp.int32)
    NBLK = PPW * (Y // NPEN)

    def blk_xy(t):
        x = sub * PPW + t // (Y // NPEN)
        y0 = (t % (Y // NPEN)) * NPEN
        return x, y0

    def stage_coords(t, pb):
        @pl.when(t % 4 == 0)
        def _stage():
            x = sub * PPW + t // (Y // NPEN)
            half = (t // 4) % 2
            drow0 = (core * X + x) * (3 * Y) + half * 48
            for comp in range(3):
                pltpu.sync_copy(defp.at[pl.ds(drow0 + comp * Y, 48), :],
                                coords_v.at[comp])

    def compute_idx(pb, t):
        ybase = (t % 4) * NPEN

        def grp_index(g, c2):
            yl = ybase + g // (Z // 16)
            zoff = (g % (Z // 16)) * 16
            xs = coords_v[0, yl, pl.ds(zoff, 16)]
            ys = coords_v[1, yl, pl.ds(zoff, 16)]
            zs = coords_v[2, yl, pl.ds(zoff, 16)]
            ix = jnp.minimum(xs.astype(jnp.int32), X - 2)
            iy = jnp.minimum(ys.astype(jnp.int32), Y - 2)
            iz = jnp.minimum(zs.astype(jnp.int32), Z - 2)
            fx = xs - ix.astype(jnp.float32)
            fy = ys - iy.astype(jnp.float32)
            fz = zs - iz.astype(jnp.float32)
            gx = 1.0 - fx
            gy = 1.0 - fy
            gz = 1.0 - fz
            v = batch_row0 + ix * _SX + iy * _SY + iz
            b16 = g * 16
            idx_v[pb, 0, pl.ds(b16, 16)] = v
            idx_v[pb, 1, pl.ds(b16, 16)] = v + _SX
            wyz = (gy * gz, gy * fz, fy * gz, fy * fz)
            for dy in (0, 1):
                for dz in (0, 1):
                    w_v[pb, dy * 2 + dz, pl.ds(b16, 16)] = gx * wyz[dy * 2 + dz]
                    w_v[pb, 4 + dy * 2 + dz, pl.ds(b16, 16)] = fx * wyz[dy * 2 + dz]
            return c2

        def grp_index2(i, c2):
            grp_index(i * 2, c2)
            grp_index(i * 2 + 1, c2)
            return c2

        lax.fori_loop(0, GRPS // 2, grp_index2, 0)

    def fire(pb, s):
        for dx in range(2):
            pltpu.async_copy(table_hbm.at[idx_v.at[pb, dx]],
                             rows_v.at[pb, dx], s)

    def drain(pb, s):
        for dx in range(2):
            pltpu.make_async_copy(table_hbm.at[idx_v.at[pb, dx]],
                                  rows_v.at[pb, dx], s).wait()

    def extract(t, pb):
        def grp_acc(g, c2):
            yl = g // (Z // 16)
            zoff = (g % (Z // 16)) * 16
            b16 = g * 16
            posv = b16 + lanes
            acc0 = jnp.zeros((16,), jnp.float32)
            acc1 = jnp.zeros((16,), jnp.float32)
            for dx in range(2):
                rv = rows_v.at[pb, dx]
                for dy in range(2):
                    for dz in range(2):
                        w = w_v[pb, dx * 4 + dy * 2 + dz, pl.ds(b16, 16)]
                        f = dy * 4 + dz * 2
                        v0 = plsc.load_gather(rv, [posv, zero + f])
                        v1 = plsc.load_gather(rv, [posv, zero + f + 1])
                        acc0 = acc0 + w * v0
                        acc1 = acc1 + w * v1
            out_v[pl.ds(yl * (2 * ZP) + zoff, 16)] = acc0
            out_v[pl.ds(yl * (2 * ZP) + ZP + zoff, 16)] = acc1
            return c2

        def grp_acc2(i, c2):
            grp_acc(i * 2, c2)
            grp_acc(i * 2 + 1, c2)
            return c2

        lax.fori_loop(0, GRPS // 2, grp_acc2, 0)
        x, y0 = blk_xy(t)
        o0 = ((core * X + x) * Y + y0) * (2 * ZP)
        pltpu.sync_copy(out_v, out_hbm.at[pl.ds(o0, NPEN * 2 * ZP)])

    stage_coords(0, 0)
    compute_idx(0, 0)
    fire(0, sem)

    def pair(tt, _):
        t0 = tt * 2
        stage_coords(t0 + 1, 1)
        compute_idx(1, t0 + 1)
        fire(1, sem2)
        drain(0, sem)
        extract(t0, 0)

        @pl.when(t0 + 2 < NBLK)
        def _prep():
            stage_coords(t0 + 2, 0)
            compute_idx(0, t0 + 2)
            fire(0, sem)

        drain(1, sem2)
        extract(t0 + 1, 1)
        return _

    lax.fori_loop(0, NBLK // 2, pair, 0)


_resample_sc = _sc_call(_resample_body)


def kernel(inputs, deformation):
    volp = jnp.pad(
        jnp.transpose(inputs, (0, 1, 2, 4, 3)).reshape(VROWS, Z),
        ((0, 0), (0, ZP - Z)))
    defp = jnp.pad(
        jnp.transpose(deformation, (0, 1, 4, 2, 3)).reshape(DROWS, Z),
        ((0, 0), (0, ZP - Z)))
    out_flat, _ = _resample_sc(volp, defp)
    out = out_flat.reshape(B, X, Y, C, ZP)[..., :Z]
    return jnp.transpose(out, (0, 1, 2, 4, 3))


# speedup vs baseline: 1.1889x; 1.0105x over previous
import functools

import jax
import jax.numpy as jnp
from jax import lax
from jax.experimental import pallas as pl
from jax.experimental.pallas import tpu as pltpu
from jax.experimental.pallas import tpu_sc as plsc

B, X, Y, Z, C = 2, 96, 96, 96, 2
ZP = 128
N = X * Y * Z
TOT = B * N
NC, NS = 2, 16
PPW = X // NS
_SX, _SY = Y * Z, Z

NPEN = 12
BL = NPEN * Z
GRPS = BL // 16
NCH = BL // 128

VROWS = B * X * Y * C
DROWS = B * X * 3 * Y
OROWS = B * X * Y


def _sc_call(body, interpret=False):
    return pl.kernel(
        body,
        out_type=(
            jax.ShapeDtypeStruct((OROWS * 2 * ZP,), jnp.float32),
            jax.ShapeDtypeStruct((TOT, 8), jnp.float32),
        ),
        mesh=plsc.VectorSubcoreMesh(core_axis_name="c", subcore_axis_name="s",
                                    num_cores=NC, num_subcores=NS),
        compiler_params=pltpu.CompilerParams(needs_layout_passes=False,
                                             use_tc_tiling_on_sc=False),
        scratch_types=[
            pltpu.VMEM((50, ZP), jnp.float32),
            pltpu.VMEM((2, 24 * Z, 8), jnp.float32),
            pltpu.VMEM((3, 48, ZP), jnp.float32),
            pltpu.VMEM((2, 2, BL), jnp.int32),
            pltpu.VMEM((2, 8, BL), jnp.float32),
            pltpu.VMEM((2, 2, BL, 8), jnp.float32),
            pltpu.VMEM((2, NPEN * 2 * ZP), jnp.float32),
            pltpu.SemaphoreType.DMA,
            pltpu.SemaphoreType.DMA,
            pltpu.SemaphoreType.DMA,
            pltpu.SemaphoreType.DMA,
            pltpu.SemaphoreType.DMA,
            pltpu.SemaphoreType.DMA,
        ],
        interpret=interpret,
    )


def _resample_body(volp, defp, out_hbm, table_hbm, stag_v, tbuf_v, coords_v,
                   idx_v, w_v, rows_v, out_v, sem, sem2, semta, semtb,
                   semoa, semob):
    core = lax.axis_index("c")
    sub = lax.axis_index("s")
    batch_row0 = core * N
    lanes = lax.iota(jnp.int32, 16)

    a_row = ((lanes % 8) // 4) * 2 + (lanes % 2)
    a_col = lanes // 8 + (lanes % 4) // 2
    t_row = lanes // 8
    t_col = lanes % 8

    def build_plane(xi, _):
        x = sub * PPW + xi
        vrow0 = (core * X + x) * (Y * C)
        for k in range(4):
            y0 = 24 * k
            tb = k % 2
            semt = (semta, semtb)[tb]
            nstage = 50 if k < 3 else 48
            pltpu.sync_copy(volp.at[pl.ds(vrow0 + 48 * k, nstage), :],
                            stag_v.at[pl.ds(0, nstage), :])

            def _wait_tb():
                pltpu.make_async_copy(
                    tbuf_v.at[tb], table_hbm.at[pl.ds(0, 24 * Z), :],
                    semt).wait()

            if k < 2:
                @pl.when(xi > 0)
                def _d():
                    _wait_tb()
            else:
                _wait_tb()

            def build_pencil(yl, _):
                rowc = a_row + 2 * yl
                trowc = t_row + yl * Z

                def build_grp(i, c2):
                    for u in range(4):
                        g = i * 4 + u
                        vals = plsc.load_gather(stag_v, [rowc, a_col + 2 * g])
                        plsc.store_scatter(tbuf_v.at[tb],
                                           [trowc + 2 * g, t_col], vals)
                    return c2

                lax.fori_loop(0, Z // 8, build_grp, 0)
                return _

            lax.fori_loop(0, 24, build_pencil, 0)
            t0 = batch_row0 + x * _SX + y0 * _SY
            pltpu.async_copy(tbuf_v.at[tb],
                             table_hbm.at[pl.ds(t0, 24 * Z), :], semt)
        return _

    lax.fori_loop(0, PPW, build_plane, 0)
    for tb in range(2):
        pltpu.make_async_copy(tbuf_v.at[tb],
                              table_hbm.at[pl.ds(0, 24 * Z), :],
                              (semta, semtb)[tb]).wait()
    plsc.subcore_barrier()

    zero = jnp.zeros((16,), jnp.int32)
    NBLK = PPW * (Y // NPEN)

    def blk_xy(t):
        x = sub * PPW + t // (Y // NPEN)
        y0 = (t % (Y // NPEN)) * NPEN
        return x, y0

    def stage_coords(t, pb):
        @pl.when(t % 4 == 0)
        def _stage():
            x = sub * PPW + t // (Y // NPEN)
            half = (t // 4) % 2
            drow0 = (core * X + x) * (3 * Y) + half * 48
            for comp in range(3):
                pltpu.sync_copy(defp.at[pl.ds(drow0 + comp * Y, 48), :],
                                coords_v.at[comp])

    def compute_idx(pb, t):
        ybase = (t % 4) * NPEN

        def grp_index(g, c2):
            yl = ybase + g // (Z // 16)
            zoff = (g % (Z // 16)) * 16
            xs = coords_v[0, yl, pl.ds(zoff, 16)]
            ys = coords_v[1, yl, pl.ds(zoff, 16)]
            zs = coords_v[2, yl, pl.ds(zoff, 16)]
            ix = jnp.minimum(xs.astype(jnp.int32), X - 2)
            iy = jnp.minimum(ys.astype(jnp.int32), Y - 2)
            iz = jnp.minimum(zs.astype(jnp.int32), Z - 2)
            fx = xs - ix.astype(jnp.float32)
            fy = ys - iy.astype(jnp.float32)
            fz = zs - iz.astype(jnp.float32)
            gx = 1.0 - fx
            gy = 1.0 - fy
            gz = 1.0 - fz
            v = batch_row0 + ix * _SX + iy * _SY + iz
            b16 = g * 16
            idx_v[pb, 0, pl.ds(b16, 16)] = v
            idx_v[pb, 1, pl.ds(b16, 16)] = v + _SX
            wyz = (gy * gz, gy * fz, fy * gz, fy * fz)
            for dy in (0, 1):
                for dz in (0, 1):
                    w_v[pb, dy * 2 + dz, pl.ds(b16, 16)] = gx * wyz[dy * 2 + dz]
                    w_v[pb, 4 + dy * 2 + dz, pl.ds(b16, 16)] = fx * wyz[dy * 2 + dz]
            return c2

        def grp_index2(i, c2):
            grp_index(i * 2, c2)
            grp_index(i * 2 + 1, c2)
            return c2

        lax.fori_loop(0, GRPS // 2, grp_index2, 0)

    def fire(pb, s):
        for dx in range(2):
            pltpu.async_copy(table_hbm.at[idx_v.at[pb, dx]],
                             rows_v.at[pb, dx], s)

    def drain(pb, s):
        for dx in range(2):
            pltpu.make_async_copy(table_hbm.at[idx_v.at[pb, dx]],
                                  rows_v.at[pb, dx], s).wait()

    def extract(t, pb):
        semo = (semoa, semob)[pb]

        @pl.when(t >= 2)
        def _wait_out():
            pltpu.make_async_copy(out_v.at[pb],
                                  out_hbm.at[pl.ds(0, NPEN * 2 * ZP)],
                                  semo).wait()

        def grp_acc(g, c2):
            yl = g // (Z // 16)
            zoff = (g % (Z // 16)) * 16
            b16 = g * 16
            posv = b16 + lanes
            acc0 = jnp.zeros((16,), jnp.float32)
            acc1 = jnp.zeros((16,), jnp.float32)
            for dx in range(2):
                rv = rows_v.at[pb, dx]
                for dy in range(2):
                    for dz in range(2):
                        w = w_v[pb, dx * 4 + dy * 2 + dz, pl.ds(b16, 16)]
                        f = dy * 4 + dz * 2
                        v0 = plsc.load_gather(rv, [posv, zero + f])
                        v1 = plsc.load_gather(rv, [posv, zero + f + 1])
                        acc0 = acc0 + w * v0
                        acc1 = acc1 + w * v1
            out_v[pb, pl.ds(yl * (2 * ZP) + zoff, 16)] = acc0
            out_v[pb, pl.ds(yl * (2 * ZP) + ZP + zoff, 16)] = acc1
            return c2

        def grp_acc2(i, c2):
            grp_acc(i * 2, c2)
            grp_acc(i * 2 + 1, c2)
            return c2

        lax.fori_loop(0, GRPS // 2, grp_acc2, 0)
        x, y0 = blk_xy(t)
        o0 = ((core * X + x) * Y + y0) * (2 * ZP)
        pltpu.async_copy(out_v.at[pb], out_hbm.at[pl.ds(o0, NPEN * 2 * ZP)],
                         semo)

    stage_coords(0, 0)
    compute_idx(0, 0)
    fire(0, sem)

    def pair(tt, _):
        t0 = tt * 2
        stage_coords(t0 + 1, 1)
        compute_idx(1, t0 + 1)
        fire(1, sem2)
        drain(0, sem)
        extract(t0, 0)

        @pl.when(t0 + 2 < NBLK)
        def _prep():
            stage_coords(t0 + 2, 0)
            compute_idx(0, t0 + 2)
            fire(0, sem)

        drain(1, sem2)
        extract(t0 + 1, 1)
        return _

    lax.fori_loop(0, NBLK // 2, pair, 0)
    for pb in range(2):
        pltpu.make_async_copy(out_v.at[pb],
                              out_hbm.at[pl.ds(0, NPEN * 2 * ZP)],
                              (semoa, semob)[pb]).wait()


_resample_sc = _sc_call(_resample_body)


def kernel(inputs, deformation):
    volp = jnp.pad(
        jnp.transpose(inputs, (0, 1, 2, 4, 3)).reshape(VROWS, Z),
        ((0, 0), (0, ZP - Z)))
    defp = jnp.pad(
        jnp.transpose(deformation, (0, 1, 4, 2, 3)).reshape(DROWS, Z),
        ((0, 0), (0, ZP - Z)))
    out_flat, _ = _resample_sc(volp, defp)
    out = out_flat.reshape(B, X, Y, C, ZP)[..., :Z]
    return jnp.transpose(out, (0, 1, 2, 4, 3))
